# initial kernel scaffold (unmeasured)
import jax
import jax.numpy as jnp
from jax import lax
from jax.experimental import pallas as pl
from jax.experimental.pallas import tpu as pltpu

M = 4096
N = 4096
KS = 2048
HM = M // 2
NC = 512
N_CHUNKS = N // NC


def kernel(A, B):
    def body(
        a_hbm, b_hbm, out_hbm,
        a_buf, b_buf, pc_buf, px_buf, ry_buf,
        a_sem, b_sems, o_sems,
        x_send, x_recv, y_send, y_recv,
        x_credit, y_credit,
    ):
        my_x = lax.axis_index("x")
        my_y = lax.axis_index("y")
        x_nbr = (1 - my_x, my_y)
        y_nbr = (my_x, 1 - my_y)

        a_cp = pltpu.make_async_copy(
            a_hbm.at[pl.ds(my_y * HM, HM), :], a_buf, a_sem
        )
        a_cp.start()

        barrier_sem = pltpu.get_barrier_semaphore()
        for nbr in (x_nbr, y_nbr):
            pl.semaphore_signal(
                barrier_sem, inc=1,
                device_id=nbr, device_id_type=pl.DeviceIdType.MESH,
            )
        pl.semaphore_wait(barrier_sem, 2)

        a_cp.wait()

        for c in range(N_CHUNKS):
            s = c % 2

            b_cp = pltpu.make_async_copy(
                b_hbm.at[:, pl.ds(c * NC, NC)], b_buf.at[s], b_sems.at[s]
            )
            b_cp.start()
            b_cp.wait()

            pc_buf[s, :, :] = jnp.dot(
                a_buf[:, :], b_buf[s], preferred_element_type=jnp.float32
            )

            if c >= 2:
                pl.semaphore_wait(x_credit, 1)
            rdma_x = pltpu.make_async_remote_copy(
                src_ref=pc_buf.at[s],
                dst_ref=px_buf.at[s],
                send_sem=x_send.at[s],
                recv_sem=x_recv.at[s],
                device_id=x_nbr,
                device_id_type=pl.DeviceIdType.MESH,
            )
            rdma_x.start()
            rdma_x.wait()

            pc_buf[s, :, :] = pc_buf[s] + px_buf[s]
            if c <= N_CHUNKS - 3:
                pl.semaphore_signal(
                    x_credit, inc=1,
                    device_id=x_nbr, device_id_type=pl.DeviceIdType.MESH,
                )

            if c >= 2:
                pl.semaphore_wait(y_credit, 1)
            rdma_y = pltpu.make_async_remote_copy(
                src_ref=pc_buf.at[s],
                dst_ref=ry_buf.at[s],
                send_sem=y_send.at[s],
                recv_sem=y_recv.at[s],
                device_id=y_nbr,
                device_id_type=pl.DeviceIdType.MESH,
            )
            rdma_y.start()
            rdma_y.wait()

            o_cp0 = pltpu.make_async_copy(
                pc_buf.at[s],
                out_hbm.at[pl.ds(my_y * HM, HM), pl.ds(c * NC, NC)],
                o_sems.at[0],
            )
            o_cp1 = pltpu.make_async_copy(
                ry_buf.at[s],
                out_hbm.at[pl.ds((1 - my_y) * HM, HM), pl.ds(c * NC, NC)],
                o_sems.at[1],
            )
            o_cp0.start()
            o_cp1.start()
            o_cp0.wait()
            o_cp1.wait()
            if c <= N_CHUNKS - 3:
                pl.semaphore_signal(
                    y_credit, inc=1,
                    device_id=y_nbr, device_id_type=pl.DeviceIdType.MESH,
                )

    return pl.pallas_call(
        body,
        out_shape=jax.ShapeDtypeStruct((M, N), jnp.float32),
        in_specs=[
            pl.BlockSpec(memory_space=pltpu.ANY),
            pl.BlockSpec(memory_space=pltpu.ANY),
        ],
        out_specs=pl.BlockSpec(memory_space=pltpu.ANY),
        scratch_shapes=[
            pltpu.VMEM((HM, KS), jnp.float32),
            pltpu.VMEM((2, KS, NC), jnp.float32),
            pltpu.VMEM((2, HM, NC), jnp.float32),
            pltpu.VMEM((2, HM, NC), jnp.float32),
            pltpu.VMEM((2, HM, NC), jnp.float32),
            pltpu.SemaphoreType.DMA,
            pltpu.SemaphoreType.DMA((2,)),
            pltpu.SemaphoreType.DMA((2,)),
            pltpu.SemaphoreType.DMA((2,)),
            pltpu.SemaphoreType.DMA((2,)),
            pltpu.SemaphoreType.DMA((2,)),
            pltpu.SemaphoreType.DMA((2,)),
            pltpu.SemaphoreType.REGULAR,
            pltpu.SemaphoreType.REGULAR,
        ],
        compiler_params=pltpu.CompilerParams(collective_id=0),
    )(A, B)


# baseline (device time: 888462 ns/iter reference)
import jax
import jax.numpy as jnp
from jax import lax
from jax.experimental import pallas as pl
from jax.experimental.pallas import tpu as pltpu

M = 4096
N = 4096
KS = 2048
HM = M // 2
NC = 512
N_CHUNKS = N // NC


def kernel(A, B):
    def body(
        a_hbm, b_hbm, out_hbm,
        a_buf, b_buf, pc_buf, px_buf, ry_buf,
        a_sem, b_sems, o_sems,
        x_send, x_recv, y_send, y_recv,
        x_credit, y_credit,
    ):
        my_x = lax.axis_index("x")
        my_y = lax.axis_index("y")
        x_nbr = (1 - my_x, my_y)
        y_nbr = (my_x, 1 - my_y)

        a_cp = pltpu.make_async_copy(
            a_hbm.at[pl.ds(my_y * HM, HM), :], a_buf, a_sem
        )
        a_cp.start()

        barrier_sem = pltpu.get_barrier_semaphore()
        for nbr in (x_nbr, y_nbr):
            pl.semaphore_signal(
                barrier_sem, inc=1,
                device_id=nbr, device_id_type=pl.DeviceIdType.MESH,
            )
        pl.semaphore_wait(barrier_sem, 2)

        a_cp.wait()

        for c in range(N_CHUNKS):
            s = c % 2

            b_cp = pltpu.make_async_copy(
                b_hbm.at[:, pl.ds(c * NC, NC)], b_buf.at[s], b_sems.at[s]
            )
            b_cp.start()
            b_cp.wait()

            pc_buf[s, :, :] = jnp.dot(
                a_buf[:, :], b_buf[s], preferred_element_type=jnp.float32
            )

            if c >= 2:
                pl.semaphore_wait(x_credit, 1)
            rdma_x = pltpu.make_async_remote_copy(
                src_ref=pc_buf.at[s],
                dst_ref=px_buf.at[s],
                send_sem=x_send.at[s],
                recv_sem=x_recv.at[s],
                device_id=x_nbr,
                device_id_type=pl.DeviceIdType.MESH,
            )
            rdma_x.start()
            rdma_x.wait()

            pc_buf[s, :, :] = pc_buf[s] + px_buf[s]
            if c <= N_CHUNKS - 3:
                pl.semaphore_signal(
                    x_credit, inc=1,
                    device_id=x_nbr, device_id_type=pl.DeviceIdType.MESH,
                )

            if c >= 2:
                pl.semaphore_wait(y_credit, 1)
            rdma_y = pltpu.make_async_remote_copy(
                src_ref=pc_buf.at[s],
                dst_ref=ry_buf.at[s],
                send_sem=y_send.at[s],
                recv_sem=y_recv.at[s],
                device_id=y_nbr,
                device_id_type=pl.DeviceIdType.MESH,
            )
            rdma_y.start()
            rdma_y.wait()

            o_cp0 = pltpu.make_async_copy(
                pc_buf.at[s],
                out_hbm.at[pl.ds(my_y * HM, HM), pl.ds(c * NC, NC)],
                o_sems.at[0],
            )
            o_cp1 = pltpu.make_async_copy(
                ry_buf.at[s],
                out_hbm.at[pl.ds((1 - my_y) * HM, HM), pl.ds(c * NC, NC)],
                o_sems.at[1],
            )
            o_cp0.start()
            o_cp1.start()
            o_cp0.wait()
            o_cp1.wait()
            if c <= N_CHUNKS - 3:
                pl.semaphore_signal(
                    y_credit, inc=1,
                    device_id=y_nbr, device_id_type=pl.DeviceIdType.MESH,
                )

    return pl.pallas_call(
        body,
        out_shape=jax.ShapeDtypeStruct((M, N), jnp.float32),
        in_specs=[
            pl.BlockSpec(memory_space=pl.ANY),
            pl.BlockSpec(memory_space=pl.ANY),
        ],
        out_specs=pl.BlockSpec(memory_space=pl.ANY),
        scratch_shapes=[
            pltpu.VMEM((HM, KS), jnp.float32),
            pltpu.VMEM((2, KS, NC), jnp.float32),
            pltpu.VMEM((2, HM, NC), jnp.float32),
            pltpu.VMEM((2, HM, NC), jnp.float32),
            pltpu.VMEM((2, HM, NC), jnp.float32),
            pltpu.SemaphoreType.DMA,
            pltpu.SemaphoreType.DMA((2,)),
            pltpu.SemaphoreType.DMA((2,)),
            pltpu.SemaphoreType.DMA((2,)),
            pltpu.SemaphoreType.DMA((2,)),
            pltpu.SemaphoreType.DMA((2,)),
            pltpu.SemaphoreType.DMA((2,)),
            pltpu.SemaphoreType.REGULAR,
            pltpu.SemaphoreType.REGULAR,
        ],
        compiler_params=pltpu.CompilerParams(
            collective_id=0,
            vmem_limit_bytes=63 * 1024 * 1024,
        ),
    )(A, B)


# device time: 470755 ns/iter; 1.8873x vs baseline; 1.8873x over previous
import jax
import jax.numpy as jnp
from jax import lax
from jax.experimental import pallas as pl
from jax.experimental.pallas import tpu as pltpu

M = 4096
N = 4096
KS = 2048
HM = M // 2
NC = 512
N_CHUNKS = N // NC


def kernel(A, B):
    def body(
        a_hbm, b_hbm, out_hbm,
        a_buf, b_buf, pc_buf, px_buf, rc_buf, ry_buf,
        a_sem, b_sems, o_sems,
        x_send, x_recv, y_send, y_recv,
        x_credit, y_credit,
    ):
        my_x = lax.axis_index("x")
        my_y = lax.axis_index("y")
        x_nbr = (1 - my_x, my_y)
        y_nbr = (my_x, 1 - my_y)

        a_cp = pltpu.make_async_copy(
            a_hbm.at[pl.ds(my_y * HM, HM), :], a_buf, a_sem
        )
        a_cp.start()

        def b_load(c):
            return pltpu.make_async_copy(
                b_hbm.at[:, pl.ds(c * NC, NC)], b_buf.at[c % 2], b_sems.at[c % 2]
            )

        b_cps = {0: b_load(0)}
        b_cps[0].start()
        if N_CHUNKS > 1:
            b_cps[1] = b_load(1)
            b_cps[1].start()

        barrier_sem = pltpu.get_barrier_semaphore()
        for nbr in (x_nbr, y_nbr):
            pl.semaphore_signal(
                barrier_sem, inc=1,
                device_id=nbr, device_id_type=pl.DeviceIdType.MESH,
            )
        pl.semaphore_wait(barrier_sem, 2)

        a_cp.wait()

        def x_desc(c):
            s = c % 2
            return pltpu.make_async_remote_copy(
                src_ref=pc_buf.at[s], dst_ref=px_buf.at[s],
                send_sem=x_send.at[s], recv_sem=x_recv.at[s],
                device_id=x_nbr, device_id_type=pl.DeviceIdType.MESH,
            )

        def y_desc(c):
            s = c % 2
            return pltpu.make_async_remote_copy(
                src_ref=rc_buf.at[s], dst_ref=ry_buf.at[s],
                send_sem=y_send.at[s], recv_sem=y_recv.at[s],
                device_id=y_nbr, device_id_type=pl.DeviceIdType.MESH,
            )

        x_rdmas = {}
        y_rdmas = {}
        for it in range(N_CHUNKS + 2):
            c = it
            if c < N_CHUNKS:
                s = c % 2
                if c >= 2:
                    x_rdmas[c - 2].wait_send()
                b_cps[c].wait()
                pc_buf[s, :, :] = jnp.dot(
                    a_buf[:, :], b_buf[s], preferred_element_type=jnp.float32
                )
                if c + 2 < N_CHUNKS:
                    b_cps[c + 2] = b_load(c + 2)
                    b_cps[c + 2].start()
                if c >= 2:
                    pl.semaphore_wait(x_credit, 1)
                x_rdmas[c] = x_desc(c)
                x_rdmas[c].start()

            d = it - 1
            if 0 <= d < N_CHUNKS:
                sd = d % 2
                x_rdmas[d].wait_recv()
                if d >= 2:
                    y_rdmas[d - 2].wait_send()
                rc_buf[sd, :, :] = pc_buf[sd] + px_buf[sd]
                if d <= N_CHUNKS - 3:
                    pl.semaphore_signal(
                        x_credit, inc=1,
                        device_id=x_nbr, device_id_type=pl.DeviceIdType.MESH,
                    )
                if d >= 2:
                    pl.semaphore_wait(y_credit, 1)
                y_rdmas[d] = y_desc(d)
                y_rdmas[d].start()

            e = it - 2
            if 0 <= e < N_CHUNKS:
                se = e % 2
                y_rdmas[e].wait_recv()
                o_cp0 = pltpu.make_async_copy(
                    rc_buf.at[se],
                    out_hbm.at[pl.ds(my_y * HM, HM), pl.ds(e * NC, NC)],
                    o_sems.at[0],
                )
                o_cp1 = pltpu.make_async_copy(
                    ry_buf.at[se],
                    out_hbm.at[pl.ds((1 - my_y) * HM, HM), pl.ds(e * NC, NC)],
                    o_sems.at[1],
                )
                o_cp0.start()
                o_cp1.start()
                o_cp0.wait()
                o_cp1.wait()
                if e <= N_CHUNKS - 3:
                    pl.semaphore_signal(
                        y_credit, inc=1,
                        device_id=y_nbr, device_id_type=pl.DeviceIdType.MESH,
                    )

        for c in (N_CHUNKS - 2, N_CHUNKS - 1):
            x_rdmas[c].wait_send()
            y_rdmas[c].wait_send()

    return pl.pallas_call(
        body,
        out_shape=jax.ShapeDtypeStruct((M, N), jnp.float32),
        in_specs=[
            pl.BlockSpec(memory_space=pl.ANY),
            pl.BlockSpec(memory_space=pl.ANY),
        ],
        out_specs=pl.BlockSpec(memory_space=pl.ANY),
        scratch_shapes=[
            pltpu.VMEM((HM, KS), jnp.float32),
            pltpu.VMEM((2, KS, NC), jnp.float32),
            pltpu.VMEM((2, HM, NC), jnp.float32),
            pltpu.VMEM((2, HM, NC), jnp.float32),
            pltpu.VMEM((2, HM, NC), jnp.float32),
            pltpu.VMEM((2, HM, NC), jnp.float32),
            pltpu.SemaphoreType.DMA,
            pltpu.SemaphoreType.DMA((2,)),
            pltpu.SemaphoreType.DMA((2,)),
            pltpu.SemaphoreType.DMA((2,)),
            pltpu.SemaphoreType.DMA((2,)),
            pltpu.SemaphoreType.DMA((2,)),
            pltpu.SemaphoreType.DMA((2,)),
            pltpu.SemaphoreType.REGULAR,
            pltpu.SemaphoreType.REGULAR,
        ],
        compiler_params=pltpu.CompilerParams(
            collective_id=0,
            vmem_limit_bytes=63 * 1024 * 1024,
        ),
    )(A, B)


# device time: 445944 ns/iter; 1.9923x vs baseline; 1.0556x over previous
import jax
import jax.numpy as jnp
from jax import lax
from jax.experimental import pallas as pl
from jax.experimental.pallas import tpu as pltpu

M = 4096
N = 4096
KS = 2048
HM = M // 2
NC = 256
N_CHUNKS = N // NC


def kernel(A, B):
    def body(
        a_hbm, b_hbm, out_hbm,
        a_buf, b_buf, pc_buf, px_buf, rc_buf, ry_buf,
        a_sem, b_sems, o_sems,
        x_send, x_recv, y_send, y_recv,
        x_credit, y_credit,
    ):
        my_x = lax.axis_index("x")
        my_y = lax.axis_index("y")
        x_nbr = (1 - my_x, my_y)
        y_nbr = (my_x, 1 - my_y)

        a_cp = pltpu.make_async_copy(
            a_hbm.at[pl.ds(my_y * HM, HM), :], a_buf, a_sem
        )
        a_cp.start()

        def b_load(c):
            return pltpu.make_async_copy(
                b_hbm.at[:, pl.ds(c * NC, NC)], b_buf.at[c % 2], b_sems.at[c % 2]
            )

        b_cps = {0: b_load(0)}
        b_cps[0].start()
        if N_CHUNKS > 1:
            b_cps[1] = b_load(1)
            b_cps[1].start()

        barrier_sem = pltpu.get_barrier_semaphore()
        for nbr in (x_nbr, y_nbr):
            pl.semaphore_signal(
                barrier_sem, inc=1,
                device_id=nbr, device_id_type=pl.DeviceIdType.MESH,
            )
        pl.semaphore_wait(barrier_sem, 2)

        a_cp.wait()

        def x_desc(c):
            s = c % 2
            return pltpu.make_async_remote_copy(
                src_ref=pc_buf.at[s], dst_ref=px_buf.at[s],
                send_sem=x_send.at[s], recv_sem=x_recv.at[s],
                device_id=x_nbr, device_id_type=pl.DeviceIdType.MESH,
            )

        def y_desc(c):
            s = c % 2
            return pltpu.make_async_remote_copy(
                src_ref=rc_buf.at[s], dst_ref=ry_buf.at[s],
                send_sem=y_send.at[s], recv_sem=y_recv.at[s],
                device_id=y_nbr, device_id_type=pl.DeviceIdType.MESH,
            )

        x_rdmas = {}
        y_rdmas = {}
        for it in range(N_CHUNKS + 2):
            c = it
            if c < N_CHUNKS:
                s = c % 2
                if c >= 2:
                    x_rdmas[c - 2].wait_send()
                b_cps[c].wait()
                pc_buf[s, :, :] = jnp.dot(
                    a_buf[:, :], b_buf[s], preferred_element_type=jnp.float32
                )
                if c + 2 < N_CHUNKS:
                    b_cps[c + 2] = b_load(c + 2)
                    b_cps[c + 2].start()
                if c >= 2:
                    pl.semaphore_wait(x_credit, 1)
                x_rdmas[c] = x_desc(c)
                x_rdmas[c].start()

            d = it - 1
            if 0 <= d < N_CHUNKS:
                sd = d % 2
                x_rdmas[d].wait_recv()
                if d >= 2:
                    y_rdmas[d - 2].wait_send()
                rc_buf[sd, :, :] = pc_buf[sd] + px_buf[sd]
                if d <= N_CHUNKS - 3:
                    pl.semaphore_signal(
                        x_credit, inc=1,
                        device_id=x_nbr, device_id_type=pl.DeviceIdType.MESH,
                    )
                if d >= 2:
                    pl.semaphore_wait(y_credit, 1)
                y_rdmas[d] = y_desc(d)
                y_rdmas[d].start()

            e = it - 2
            if 0 <= e < N_CHUNKS:
                se = e % 2
                y_rdmas[e].wait_recv()
                o_cp0 = pltpu.make_async_copy(
                    rc_buf.at[se],
                    out_hbm.at[pl.ds(my_y * HM, HM), pl.ds(e * NC, NC)],
                    o_sems.at[0],
                )
                o_cp1 = pltpu.make_async_copy(
                    ry_buf.at[se],
                    out_hbm.at[pl.ds((1 - my_y) * HM, HM), pl.ds(e * NC, NC)],
                    o_sems.at[1],
                )
                o_cp0.start()
                o_cp1.start()
                o_cp0.wait()
                o_cp1.wait()
                if e <= N_CHUNKS - 3:
                    pl.semaphore_signal(
                        y_credit, inc=1,
                        device_id=y_nbr, device_id_type=pl.DeviceIdType.MESH,
                    )

        for c in (N_CHUNKS - 2, N_CHUNKS - 1):
            x_rdmas[c].wait_send()
            y_rdmas[c].wait_send()

    return pl.pallas_call(
        body,
        out_shape=jax.ShapeDtypeStruct((M, N), jnp.float32),
        in_specs=[
            pl.BlockSpec(memory_space=pl.ANY),
            pl.BlockSpec(memory_space=pl.ANY),
        ],
        out_specs=pl.BlockSpec(memory_space=pl.ANY),
        scratch_shapes=[
            pltpu.VMEM((HM, KS), jnp.float32),
            pltpu.VMEM((2, KS, NC), jnp.float32),
            pltpu.VMEM((2, HM, NC), jnp.float32),
            pltpu.VMEM((2, HM, NC), jnp.float32),
            pltpu.VMEM((2, HM, NC), jnp.float32),
            pltpu.VMEM((2, HM, NC), jnp.float32),
            pltpu.SemaphoreType.DMA,
            pltpu.SemaphoreType.DMA((2,)),
            pltpu.SemaphoreType.DMA((2,)),
            pltpu.SemaphoreType.DMA((2,)),
            pltpu.SemaphoreType.DMA((2,)),
            pltpu.SemaphoreType.DMA((2,)),
            pltpu.SemaphoreType.DMA((2,)),
            pltpu.SemaphoreType.REGULAR,
            pltpu.SemaphoreType.REGULAR,
        ],
        compiler_params=pltpu.CompilerParams(
            collective_id=0,
            vmem_limit_bytes=63 * 1024 * 1024,
        ),
    )(A, B)
